# baseline (speedup 1.0000x reference)
import functools

import jax
import jax.numpy as jnp
from jax import lax
from jax.experimental import pallas as pl
from jax.experimental.pallas import tpu as pltpu
from jax.experimental.pallas import tpu_sc as plsc

NCB = 16
CB_DIM = 4
CB_SIZE = 1024
_INV_LN2 = 1.4426950408889634
_AUG_D = 16
_NW = 32


def _vq_kernel(lam_ref, xs_ref, cbTn_ref, logits_ref, gidx_ref):
    c = pl.program_id(0)

    xs = xs_ref[0]
    cbTn = cbTn_ref[0]
    lg = logits_ref[0]

    m = jnp.max(lg, axis=-1, keepdims=True)
    lse = jnp.log(jnp.sum(jnp.exp(lg - m), axis=-1, keepdims=True)) + m
    log2p = (lse - lg) * _INV_LN2
    rate_bias = log2p / lam_ref[0, 0]

    xn = jnp.sum(xs * xs, axis=0)[None, :]
    cbn = 0.25 * jnp.sum(cbTn * cbTn, axis=0)[None, :]

    cbnT = cbn.reshape(CB_SIZE, 1)
    rate_biasT = rate_bias.reshape(CB_SIZE, 1)

    prodT = jax.lax.dot_general(cbTn, xs, (((0,), (0,)), ((), ())),
                                preferred_element_type=jnp.float32)
    dist = rate_biasT + ((xn + cbnT) + prodT)

    idx = jnp.argmin(dist, axis=0).astype(jnp.int32)
    gidx_ref[0, 0] = idx + c * CB_SIZE


def _sc_gather(gidx_hbm, aug_hbm, rows_hbm, acc_hbm, idx_v, rows_v, accv, sem):
    rows_per_tile = idx_v.shape[0]
    wid = lax.axis_index("s") * 2 + lax.axis_index("c")
    base = wid * rows_per_tile
    pltpu.sync_copy(gidx_hbm.at[pl.ds(base, rows_per_tile)], idx_v)
    pltpu.async_copy(aug_hbm.at[idx_v], rows_v, sem).wait()
    pltpu.sync_copy(rows_v, rows_hbm.at[pl.ds(base, rows_per_tile)])

    def body(i, acc):
        return acc + rows_v[i]
    accv[...] = lax.fori_loop(0, rows_per_tile, body,
                              jnp.zeros((_AUG_D,), jnp.float32))
    pltpu.sync_copy(accv, acc_hbm.at[wid])


def kernel(x, codebook, logits, lmbda):
    shape = x.shape
    xf = x.reshape(-1, NCB, CB_DIM)
    n = xf.shape[0]
    nb = min(n, 2048)
    nblk = n // nb

    xs = xf.transpose(1, 2, 0)
    cbTn = codebook.transpose(0, 2, 1) * (-2.0)
    lg3 = logits.reshape(NCB, 1, CB_SIZE)
    lam = jnp.asarray(lmbda, jnp.float32).reshape(1, 1)
    log2p_col = jax.nn.log_softmax(logits, axis=-1) * (-_INV_LN2)
    aug = jnp.concatenate(
        [codebook, log2p_col[..., None],
         jnp.zeros((NCB, CB_SIZE, _AUG_D - CB_DIM - 1), jnp.float32)],
        axis=-1).reshape(NCB * CB_SIZE, _AUG_D)

    gidx = pl.pallas_call(
        _vq_kernel,
        grid=(NCB, nblk),
        in_specs=[
            pl.BlockSpec(memory_space=pltpu.SMEM),
            pl.BlockSpec((1, CB_DIM, nb), lambda c, b: (c, 0, b)),
            pl.BlockSpec((1, CB_DIM, CB_SIZE), lambda c, b: (c, 0, 0)),
            pl.BlockSpec((1, 1, CB_SIZE), lambda c, b: (c, 0, 0)),
        ],
        out_specs=pl.BlockSpec((1, 1, nb), lambda c, b: (c, 0, b)),
        out_shape=jax.ShapeDtypeStruct((NCB, 1, n), jnp.int32),
    )(lam, xs, cbTn, lg3)

    total = NCB * n
    rpt = total // _NW
    mesh = plsc.VectorSubcoreMesh(core_axis_name="c", subcore_axis_name="s")
    sc = functools.partial(
        pl.kernel, mesh=mesh,
        out_type=[
            jax.ShapeDtypeStruct((total, _AUG_D), jnp.float32),
            jax.ShapeDtypeStruct((_NW, _AUG_D), jnp.float32),
        ],
        scratch_types=[
            pltpu.VMEM((rpt,), jnp.int32),
            pltpu.VMEM((rpt, _AUG_D), jnp.float32),
            pltpu.VMEM((_AUG_D,), jnp.float32),
            pltpu.SemaphoreType.DMA,
        ],
        compiler_params=pltpu.CompilerParams(use_tc_tiling_on_sc=False),
    )(_sc_gather)
    rows, acc = sc(gidx.reshape(total), aug)

    x_hat = rows.reshape(NCB, n, _AUG_D)[:, :, :CB_DIM].transpose(1, 0, 2)
    rate_uem = jnp.sum(acc[:, CB_DIM])
    zero = jnp.zeros((1,), dtype=jnp.float32)
    return (x_hat.reshape(shape), rate_uem, jnp.zeros_like(rate_uem), zero, zero)


# speedup vs baseline: 1.0276x; 1.0276x over previous
import functools

import jax
import jax.numpy as jnp
from jax import lax
from jax.experimental import pallas as pl
from jax.experimental.pallas import tpu as pltpu
from jax.experimental.pallas import tpu_sc as plsc

NCB = 16
CB_DIM = 4
CB_SIZE = 1024
_INV_LN2 = 1.4426950408889634
_AUG_D = 16
_NW = 32


def _vq_kernel(lam_ref, xs_ref, cbTn_ref, logits_ref, gidx_ref):
    c = pl.program_id(0)

    xs = xs_ref[0]
    cbTn = cbTn_ref[0]
    lg = logits_ref[0]

    m = jnp.max(lg, axis=-1, keepdims=True)
    lse = jnp.log(jnp.sum(jnp.exp(lg - m), axis=-1, keepdims=True)) + m
    log2p = (lse - lg) * _INV_LN2
    rate_bias = log2p / lam_ref[0, 0]

    xn = jnp.sum(xs * xs, axis=0)[None, :]
    cbn = 0.25 * jnp.sum(cbTn * cbTn, axis=0)[None, :]

    cbnT = cbn.reshape(CB_SIZE, 1)
    rate_biasT = rate_bias.reshape(CB_SIZE, 1)

    prodT = jax.lax.dot_general(cbTn, xs, (((0,), (0,)), ((), ())),
                                preferred_element_type=jnp.float32)
    dist = rate_biasT + ((xn + cbnT) + prodT)

    idx = jnp.argmin(dist, axis=0).astype(jnp.int32)
    gidx_ref[0, 0] = idx + c * CB_SIZE


def _sc_gather(gidx_hbm, aug_hbm, rows_hbm, acc_hbm, idx_v, rows_v, accv, sem):
    rows_per_tile = idx_v.shape[0]
    wid = lax.axis_index("s") * 2 + lax.axis_index("c")
    base = wid * rows_per_tile
    pltpu.sync_copy(gidx_hbm.at[pl.ds(base, rows_per_tile)], idx_v)
    pltpu.async_copy(aug_hbm.at[idx_v], rows_v, sem).wait()
    pltpu.sync_copy(rows_v, rows_hbm.at[pl.ds(base, rows_per_tile)])

    def body(i, acc):
        return acc + rows_v[i]
    accv[...] = lax.fori_loop(0, rows_per_tile, body,
                              jnp.zeros((_AUG_D,), jnp.float32))
    pltpu.sync_copy(accv, acc_hbm.at[wid])


def kernel(x, codebook, logits, lmbda):
    shape = x.shape
    xf = x.reshape(-1, NCB, CB_DIM)
    n = xf.shape[0]
    nb = min(n, 4096)
    nblk = n // nb

    xs = xf.transpose(1, 2, 0)
    cbTn = codebook.transpose(0, 2, 1) * (-2.0)
    lg3 = logits.reshape(NCB, 1, CB_SIZE)
    lam = jnp.asarray(lmbda, jnp.float32).reshape(1, 1)
    log2p_col = jax.nn.log_softmax(logits, axis=-1) * (-_INV_LN2)
    aug = jnp.concatenate(
        [codebook, log2p_col[..., None],
         jnp.zeros((NCB, CB_SIZE, _AUG_D - CB_DIM - 1), jnp.float32)],
        axis=-1).reshape(NCB * CB_SIZE, _AUG_D)

    nh = n // 2

    def tc_stage(xs_h):
        return pl.pallas_call(
            _vq_kernel,
            grid=(NCB, 1),
            in_specs=[
                pl.BlockSpec(memory_space=pltpu.SMEM),
                pl.BlockSpec((1, CB_DIM, nh), lambda c, b: (c, 0, b)),
                pl.BlockSpec((1, CB_DIM, CB_SIZE), lambda c, b: (c, 0, 0)),
                pl.BlockSpec((1, 1, CB_SIZE), lambda c, b: (c, 0, 0)),
            ],
            out_specs=pl.BlockSpec((1, 1, nh), lambda c, b: (c, 0, b)),
            out_shape=jax.ShapeDtypeStruct((NCB, 1, nh), jnp.int32),
        )(lam, xs_h, cbTn, lg3)

    total = NCB * nh
    rpt = total // _NW
    mesh = plsc.VectorSubcoreMesh(core_axis_name="c", subcore_axis_name="s")
    sc = functools.partial(
        pl.kernel, mesh=mesh,
        out_type=[
            jax.ShapeDtypeStruct((total, _AUG_D), jnp.float32),
            jax.ShapeDtypeStruct((_NW, _AUG_D), jnp.float32),
        ],
        scratch_types=[
            pltpu.VMEM((rpt,), jnp.int32),
            pltpu.VMEM((rpt, _AUG_D), jnp.float32),
            pltpu.VMEM((_AUG_D,), jnp.float32),
            pltpu.SemaphoreType.DMA,
        ],
        compiler_params=pltpu.CompilerParams(use_tc_tiling_on_sc=False),
    )(_sc_gather)

    gidx0 = tc_stage(xs[:, :, :nh])
    gidx1 = tc_stage(xs[:, :, nh:])
    rows0, acc0 = sc(gidx0.reshape(total), aug)
    rows1, acc1 = sc(gidx1.reshape(total), aug)

    xh0 = rows0.reshape(NCB, nh, _AUG_D)[:, :, :CB_DIM]
    xh1 = rows1.reshape(NCB, nh, _AUG_D)[:, :, :CB_DIM]
    x_hat = jnp.concatenate([xh0, xh1], axis=1).transpose(1, 0, 2)
    rate_uem = jnp.sum(acc0[:, CB_DIM]) + jnp.sum(acc1[:, CB_DIM])
    zero = jnp.zeros((1,), dtype=jnp.float32)
    return (x_hat.reshape(shape), rate_uem, jnp.zeros_like(rate_uem), zero, zero)


# speedup vs baseline: 1.0644x; 1.0358x over previous
import functools

import jax
import jax.numpy as jnp
from jax import lax
from jax.experimental import pallas as pl
from jax.experimental.pallas import tpu as pltpu
from jax.experimental.pallas import tpu_sc as plsc

NCB = 16
CB_DIM = 4
CB_SIZE = 1024
_INV_LN2 = 1.4426950408889634
_AUG_D = 16
_NW = 32


def _vq_kernel(lam_ref, xs_ref, cbTn_ref, logits_ref, gidx_ref):
    c = pl.program_id(0)

    xs = xs_ref[0]
    cbTn = cbTn_ref[0]
    lg = logits_ref[0]

    m = jnp.max(lg, axis=-1, keepdims=True)
    lse = jnp.log(jnp.sum(jnp.exp(lg - m), axis=-1, keepdims=True)) + m
    log2p = (lse - lg) * _INV_LN2
    rate_bias = log2p / lam_ref[0, 0]

    xn = jnp.sum(xs * xs, axis=0)[None, :]
    cbn = 0.25 * jnp.sum(cbTn * cbTn, axis=0)[None, :]

    cbnT = cbn.reshape(CB_SIZE, 1)
    rate_biasT = rate_bias.reshape(CB_SIZE, 1)

    prodT = jax.lax.dot_general(cbTn, xs, (((0,), (0,)), ((), ())),
                                preferred_element_type=jnp.float32)
    dist = rate_biasT + ((xn + cbnT) + prodT)

    idx = jnp.argmin(dist, axis=0).astype(jnp.int32)
    gidx_ref[0, 0] = idx + c * CB_SIZE


def _sc_gather(gidx_hbm, aug_hbm, rows_hbm, acc_hbm, idx_v, rows_v, accv, sem):
    rows_per_tile = idx_v.shape[0]
    wid = lax.axis_index("s") * 2 + lax.axis_index("c")
    base = wid * rows_per_tile
    pltpu.sync_copy(gidx_hbm.at[pl.ds(base, rows_per_tile)], idx_v)
    pltpu.async_copy(aug_hbm.at[idx_v], rows_v, sem).wait()
    pltpu.sync_copy(rows_v, rows_hbm.at[pl.ds(base, rows_per_tile)])

    def body(i, acc):
        b = i * 8
        for j in range(8):
            acc = acc + rows_v[b + j]
        return acc
    accv[...] = lax.fori_loop(0, rows_per_tile // 8, body,
                              jnp.zeros((_AUG_D,), jnp.float32))
    pltpu.sync_copy(accv, acc_hbm.at[wid])


def kernel(x, codebook, logits, lmbda):
    shape = x.shape
    xf = x.reshape(-1, NCB, CB_DIM)
    n = xf.shape[0]
    nb = min(n, 4096)
    nblk = n // nb

    xs = xf.transpose(1, 2, 0)
    cbTn = codebook.transpose(0, 2, 1) * (-2.0)
    lg3 = logits.reshape(NCB, 1, CB_SIZE)
    lam = jnp.asarray(lmbda, jnp.float32).reshape(1, 1)
    log2p_col = jax.nn.log_softmax(logits, axis=-1) * (-_INV_LN2)
    aug = jnp.concatenate(
        [codebook, log2p_col[..., None],
         jnp.zeros((NCB, CB_SIZE, _AUG_D - CB_DIM - 1), jnp.float32)],
        axis=-1).reshape(NCB * CB_SIZE, _AUG_D)

    gidx = pl.pallas_call(
        _vq_kernel,
        grid=(NCB, nblk),
        in_specs=[
            pl.BlockSpec(memory_space=pltpu.SMEM),
            pl.BlockSpec((1, CB_DIM, nb), lambda c, b: (c, 0, b)),
            pl.BlockSpec((1, CB_DIM, CB_SIZE), lambda c, b: (c, 0, 0)),
            pl.BlockSpec((1, 1, CB_SIZE), lambda c, b: (c, 0, 0)),
        ],
        out_specs=pl.BlockSpec((1, 1, nb), lambda c, b: (c, 0, b)),
        out_shape=jax.ShapeDtypeStruct((NCB, 1, n), jnp.int32),
    )(lam, xs, cbTn, lg3)

    total = NCB * n
    rpt = total // _NW
    mesh = plsc.VectorSubcoreMesh(core_axis_name="c", subcore_axis_name="s")
    sc = functools.partial(
        pl.kernel, mesh=mesh,
        out_type=[
            jax.ShapeDtypeStruct((total, _AUG_D), jnp.float32),
            jax.ShapeDtypeStruct((_NW, _AUG_D), jnp.float32),
        ],
        scratch_types=[
            pltpu.VMEM((rpt,), jnp.int32),
            pltpu.VMEM((rpt, _AUG_D), jnp.float32),
            pltpu.VMEM((_AUG_D,), jnp.float32),
            pltpu.SemaphoreType.DMA,
        ],
        compiler_params=pltpu.CompilerParams(use_tc_tiling_on_sc=False),
    )(_sc_gather)
    rows, acc = sc(gidx.reshape(total), aug)

    x_hat = rows.reshape(NCB, n, _AUG_D)[:, :, :CB_DIM].transpose(1, 0, 2)
    rate_uem = jnp.sum(acc[:, CB_DIM])
    zero = jnp.zeros((1,), dtype=jnp.float32)
    return (x_hat.reshape(shape), rate_uem, jnp.zeros_like(rate_uem), zero, zero)


# speedup vs baseline: 1.0725x; 1.0077x over previous
import functools

import jax
import jax.numpy as jnp
from jax import lax
from jax.experimental import pallas as pl
from jax.experimental.pallas import tpu as pltpu
from jax.experimental.pallas import tpu_sc as plsc

NCB = 16
CB_DIM = 4
CB_SIZE = 1024
_INV_LN2 = 1.4426950408889634
_AUG_D = 16
_NW = 32


def _vq_kernel(lam_ref, xs_ref, cbTn_ref, logits_ref, gidx_ref):
    c = pl.program_id(0)

    xs = xs_ref[0]
    cbTn = cbTn_ref[0]
    lg = logits_ref[0]

    m = jnp.max(lg, axis=-1, keepdims=True)
    lse = jnp.log(jnp.sum(jnp.exp(lg - m), axis=-1, keepdims=True)) + m
    log2p = (lse - lg) * _INV_LN2
    rate_bias = log2p / lam_ref[0, 0]

    xn = jnp.sum(xs * xs, axis=0)[None, :]
    cbn = 0.25 * jnp.sum(cbTn * cbTn, axis=0)[None, :]

    cbnT = cbn.reshape(CB_SIZE, 1)
    rate_biasT = rate_bias.reshape(CB_SIZE, 1)

    prodT = jax.lax.dot_general(cbTn, xs, (((0,), (0,)), ((), ())),
                                preferred_element_type=jnp.float32)
    dist = rate_biasT + ((xn + cbnT) + prodT)

    idx = jnp.argmin(dist, axis=0).astype(jnp.int32)
    gidx_ref[0, 0] = idx + c * CB_SIZE


def _sc_gather(gidx_hbm, aug_hbm, rows_hbm, acc_hbm, idx_v, rows_v, accv, sem):
    rows_per_tile = idx_v.shape[0]
    wid = lax.axis_index("s") * 2 + lax.axis_index("c")
    base = wid * rows_per_tile
    pltpu.sync_copy(gidx_hbm.at[pl.ds(base, rows_per_tile)], idx_v)
    pltpu.async_copy(aug_hbm.at[idx_v], rows_v, sem).wait()
    pltpu.sync_copy(rows_v, rows_hbm.at[pl.ds(base, rows_per_tile)])

    def body(i, accs):
        b = i * 16
        return tuple(
            accs[k] + rows_v[b + k] + rows_v[b + 4 + k]
            + rows_v[b + 8 + k] + rows_v[b + 12 + k]
            for k in range(4)
        )
    z = jnp.zeros((_AUG_D,), jnp.float32)
    a0, a1, a2, a3 = lax.fori_loop(0, rows_per_tile // 16, body,
                                   (z, z, z, z))
    accv[...] = (a0 + a1) + (a2 + a3)
    pltpu.sync_copy(accv, acc_hbm.at[wid])


def kernel(x, codebook, logits, lmbda):
    shape = x.shape
    xf = x.reshape(-1, NCB, CB_DIM)
    n = xf.shape[0]
    nb = min(n, 4096)
    nblk = n // nb

    xs = xf.transpose(1, 2, 0)
    cbTn = codebook.transpose(0, 2, 1) * (-2.0)
    lg3 = logits.reshape(NCB, 1, CB_SIZE)
    lam = jnp.asarray(lmbda, jnp.float32).reshape(1, 1)
    log2p_col = jax.nn.log_softmax(logits, axis=-1) * (-_INV_LN2)
    aug = jnp.concatenate(
        [codebook, log2p_col[..., None],
         jnp.zeros((NCB, CB_SIZE, _AUG_D - CB_DIM - 1), jnp.float32)],
        axis=-1).reshape(NCB * CB_SIZE, _AUG_D)

    gidx = pl.pallas_call(
        _vq_kernel,
        grid=(NCB, nblk),
        in_specs=[
            pl.BlockSpec(memory_space=pltpu.SMEM),
            pl.BlockSpec((1, CB_DIM, nb), lambda c, b: (c, 0, b)),
            pl.BlockSpec((1, CB_DIM, CB_SIZE), lambda c, b: (c, 0, 0)),
            pl.BlockSpec((1, 1, CB_SIZE), lambda c, b: (c, 0, 0)),
        ],
        out_specs=pl.BlockSpec((1, 1, nb), lambda c, b: (c, 0, b)),
        out_shape=jax.ShapeDtypeStruct((NCB, 1, n), jnp.int32),
    )(lam, xs, cbTn, lg3)

    total = NCB * n
    rpt = total // _NW
    mesh = plsc.VectorSubcoreMesh(core_axis_name="c", subcore_axis_name="s")
    sc = functools.partial(
        pl.kernel, mesh=mesh,
        out_type=[
            jax.ShapeDtypeStruct((total, _AUG_D), jnp.float32),
            jax.ShapeDtypeStruct((_NW, _AUG_D), jnp.float32),
        ],
        scratch_types=[
            pltpu.VMEM((rpt,), jnp.int32),
            pltpu.VMEM((rpt, _AUG_D), jnp.float32),
            pltpu.VMEM((_AUG_D,), jnp.float32),
            pltpu.SemaphoreType.DMA,
        ],
        compiler_params=pltpu.CompilerParams(use_tc_tiling_on_sc=False),
    )(_sc_gather)
    rows, acc = sc(gidx.reshape(total), aug)

    x_hat = rows.reshape(NCB, n, _AUG_D)[:, :, :CB_DIM].transpose(1, 0, 2)
    rate_uem = jnp.sum(acc[:, CB_DIM])
    zero = jnp.zeros((1,), dtype=jnp.float32)
    return (x_hat.reshape(shape), rate_uem, jnp.zeros_like(rate_uem), zero, zero)
